# Initial kernel scaffold; baseline (speedup 1.0000x reference)
import jax
import jax.numpy as jnp
from jax.experimental import pallas as pl


def kernel(x_user, x_restaurant, edge_label_index):
    raise NotImplementedError("write your pallas kernel here")


# speedup vs baseline: 2.8782x; 2.8782x over previous
import functools

import jax
import jax.numpy as jnp
from jax import lax
from jax.experimental import pallas as pl
from jax.experimental.pallas import tpu as pltpu
from jax.experimental.pallas import tpu_sc as plsc

D = 128
L = 16
C = 80


def _make_sc_kernel(E, NC, NS):
    NW = NC * NS
    per_w = E // NW
    n_chunks = per_w // C
    mesh = plsc.VectorSubcoreMesh(core_axis_name="c", subcore_axis_name="s")

    @functools.partial(
        pl.kernel,
        mesh=mesh,
        out_type=jax.ShapeDtypeStruct((E,), jnp.float32),
        scratch_types=[
            pltpu.VMEM((C,), jnp.int32),
            pltpu.VMEM((C,), jnp.int32),
            pltpu.VMEM((C, D), jnp.float32),
            pltpu.VMEM((C, D), jnp.float32),
            pltpu.VMEM((C,), jnp.float32),
            pltpu.SemaphoreType.DMA,
        ],
    )
    def k(xu_hbm, xr_hbm, i0_hbm, i1_hbm, out_hbm, i0_v, i1_v, u_v, r_v, o_v, sem):
        wid = lax.axis_index("s") * NC + lax.axis_index("c")
        base = wid * per_w
        lane = lax.iota(jnp.int32, L)
        perms = [lane ^ (1 << t) for t in range(4)]

        def chunk_body(j, carry):
            off = base + j * C
            pltpu.sync_copy(i0_hbm.at[pl.ds(off, C)], i0_v)
            pltpu.sync_copy(i1_hbm.at[pl.ds(off, C)], i1_v)
            cp_u = pltpu.async_copy(xu_hbm.at[i0_v], u_v, sem)
            cp_r = pltpu.async_copy(xr_hbm.at[i1_v], r_v, sem)
            cp_u.wait()
            cp_r.wait()

            def zero_body(g, c2):
                o_v[pl.ds(g * L, L)] = jnp.zeros((L,), jnp.float32)
                return c2

            lax.fori_loop(0, C // L, zero_body, 0)

            def edge_body(e, c2):
                p = [u_v[e, pl.ds(kk * L, L)] * r_v[e, pl.ds(kk * L, L)]
                     for kk in range(D // L)]
                acc = ((p[0] + p[1]) + (p[2] + p[3])) + \
                      ((p[4] + p[5]) + (p[6] + p[7]))
                for pm in perms:
                    acc = acc + acc.at[pm].get(mode="promise_in_bounds")
                m = e % L
                contrib = jnp.where(lane == m, acc, 0.0)
                plsc.addupdate(o_v.at[pl.ds(e - m, L)], contrib)
                return c2

            lax.fori_loop(0, C, edge_body, 0)
            pltpu.sync_copy(o_v, out_hbm.at[pl.ds(off, C)])
            return carry

        lax.fori_loop(0, n_chunks, chunk_body, 0)

    return k


def kernel(x_user, x_restaurant, edge_label_index):
    E = edge_label_index.shape[1]
    idx0 = edge_label_index[0].astype(jnp.int32)
    idx1 = edge_label_index[1].astype(jnp.int32)
    info = plsc.get_sparse_core_info()
    sc = _make_sc_kernel(E, info.num_cores, info.num_subcores)
    return sc(x_user, x_restaurant, idx0, idx1)


# speedup vs baseline: 5.6241x; 1.9540x over previous
import functools

import jax
import jax.numpy as jnp
from jax import lax
from jax.experimental import pallas as pl
from jax.experimental.pallas import tpu as pltpu
from jax.experimental.pallas import tpu_sc as plsc

D = 128
L = 16
C = 80
NBUF = 4


def _make_sc_kernel(E, NC, NS):
    NW = NC * NS
    per_w = E // NW
    n_chunks = per_w // C
    n_outer = (n_chunks + NBUF - 1) // NBUF
    mesh = plsc.VectorSubcoreMesh(core_axis_name="c", subcore_axis_name="s")

    @functools.partial(
        pl.kernel,
        mesh=mesh,
        out_type=jax.ShapeDtypeStruct((E,), jnp.float32),
        scratch_types=(
            [pltpu.VMEM((per_w,), jnp.int32) for _ in range(2)]
            + [pltpu.VMEM((C, D), jnp.float32) for _ in range(2 * NBUF)]
            + [pltpu.VMEM((per_w,), jnp.float32)]
            + [pltpu.SemaphoreType.DMA for _ in range(NBUF)]
        ),
    )
    def k(xu_hbm, xr_hbm, i0_hbm, i1_hbm, out_hbm, *refs):
        i0_v, i1_v = refs[0], refs[1]
        u_bufs = refs[2:2 + NBUF]
        r_bufs = refs[2 + NBUF:2 + 2 * NBUF]
        o_v = refs[2 + 2 * NBUF]
        sems = refs[3 + 2 * NBUF:3 + 2 * NBUF + NBUF]

        wid = lax.axis_index("s") * NC + lax.axis_index("c")
        base = wid * per_w
        lane = lax.iota(jnp.int32, L)
        perms = [lane ^ (1 << t) for t in range(4)]

        pltpu.sync_copy(i0_hbm.at[pl.ds(base, per_w)], i0_v)
        pltpu.sync_copy(i1_hbm.at[pl.ds(base, per_w)], i1_v)

        def zero_body(g, c2):
            o_v[pl.ds(g * L, L)] = jnp.zeros((L,), jnp.float32)
            return c2

        lax.fori_loop(0, per_w // L, zero_body, 0)

        def start_gathers(j, b):
            pltpu.async_copy(xu_hbm.at[i0_v.at[pl.ds(j * C, C)]], u_bufs[b], sems[b])
            pltpu.async_copy(xr_hbm.at[i1_v.at[pl.ds(j * C, C)]], r_bufs[b], sems[b])

        def wait_gathers(j, b):
            pltpu.make_async_copy(xu_hbm.at[i0_v.at[pl.ds(j * C, C)]], u_bufs[b], sems[b]).wait()
            pltpu.make_async_copy(xr_hbm.at[i1_v.at[pl.ds(j * C, C)]], r_bufs[b], sems[b]).wait()

        def compute_chunk(j, b):
            u_v, r_v = u_bufs[b], r_bufs[b]
            out0 = j * C

            def edge_body(e, c2):
                p = [u_v[e, pl.ds(kk * L, L)] * r_v[e, pl.ds(kk * L, L)]
                     for kk in range(D // L)]
                acc = ((p[0] + p[1]) + (p[2] + p[3])) + \
                      ((p[4] + p[5]) + (p[6] + p[7]))
                for pm in perms:
                    acc = acc + acc.at[pm].get(mode="promise_in_bounds")
                m = e % L
                contrib = jnp.where(lane == m, acc, 0.0)
                plsc.addupdate(o_v.at[pl.ds(out0 + e - m, L)], contrib)
                return c2

            lax.fori_loop(0, C, edge_body, 0)

        for b in range(NBUF - 1):
            if b < n_chunks:
                start_gathers(b, b)

        def outer_body(g, c2):
            for b in range(NBUF):
                j = g * NBUF + b
                jn = j + NBUF - 1
                sb = b
                nb = (b + NBUF - 1) % NBUF

                @pl.when(jn < n_chunks)
                def _():
                    start_gathers(jn, nb)

                @pl.when(j < n_chunks)
                def _():
                    wait_gathers(j, sb)
                    compute_chunk(j, sb)

            return c2

        lax.fori_loop(0, n_outer, outer_body, 0)

        pltpu.sync_copy(o_v, out_hbm.at[pl.ds(base, per_w)])

    return k


def kernel(x_user, x_restaurant, edge_label_index):
    E = edge_label_index.shape[1]
    info = plsc.get_sparse_core_info()
    NC, NS = info.num_cores, info.num_subcores
    per_w = E // (NC * NS)
    n_chunks = per_w // C
    idx0 = edge_label_index[0].astype(jnp.int32)
    idx1 = edge_label_index[1].astype(jnp.int32)
    sc = _make_sc_kernel(E, NC, NS)
    return sc(x_user, x_restaurant, idx0, idx1)


# speedup vs baseline: 9.5795x; 1.7033x over previous
import functools

import jax
import jax.numpy as jnp
from jax import lax
from jax.experimental import pallas as pl
from jax.experimental.pallas import tpu as pltpu
from jax.experimental.pallas import tpu_sc as plsc

D = 128
L = 16
C = 80
NBUF = 4


def _make_sc_kernel(E, NC, NS):
    NW = NC * NS
    per_w = E // NW
    n_chunks = per_w // C
    n_outer = (n_chunks + NBUF - 1) // NBUF
    mesh = plsc.VectorSubcoreMesh(core_axis_name="c", subcore_axis_name="s")

    @functools.partial(
        pl.kernel,
        mesh=mesh,
        out_type=jax.ShapeDtypeStruct((E,), jnp.float32),
        scratch_types=(
            [pltpu.VMEM((per_w,), jnp.int32) for _ in range(2)]
            + [pltpu.VMEM((C, D), jnp.float32) for _ in range(2 * NBUF)]
            + [pltpu.VMEM((per_w,), jnp.float32)]
            + [pltpu.SemaphoreType.DMA for _ in range(NBUF)]
        ),
    )
    def k(xu_hbm, xr_hbm, i0_hbm, i1_hbm, out_hbm, *refs):
        i0_v, i1_v = refs[0], refs[1]
        u_bufs = refs[2:2 + NBUF]
        r_bufs = refs[2 + NBUF:2 + 2 * NBUF]
        o_v = refs[2 + 2 * NBUF]
        sems = refs[3 + 2 * NBUF:3 + 2 * NBUF + NBUF]

        wid = lax.axis_index("s") * NC + lax.axis_index("c")
        base = wid * per_w
        lane = lax.iota(jnp.int32, L)
        perms = [lane ^ (1 << t) for t in range(4)]

        pltpu.sync_copy(i0_hbm.at[pl.ds(base, per_w)], i0_v)
        pltpu.sync_copy(i1_hbm.at[pl.ds(base, per_w)], i1_v)

        def zero_body(g, c2):
            o_v[pl.ds(g * L, L)] = jnp.zeros((L,), jnp.float32)
            return c2

        lax.fori_loop(0, per_w // L, zero_body, 0)

        def start_gathers(j, b):
            pltpu.async_copy(xu_hbm.at[i0_v.at[pl.ds(j * C, C)]], u_bufs[b], sems[b])
            pltpu.async_copy(xr_hbm.at[i1_v.at[pl.ds(j * C, C)]], r_bufs[b], sems[b])

        def wait_gathers(j, b):
            pltpu.make_async_copy(xu_hbm.at[i0_v.at[pl.ds(j * C, C)]], u_bufs[b], sems[b]).wait()
            pltpu.make_async_copy(xr_hbm.at[i1_v.at[pl.ds(j * C, C)]], r_bufs[b], sems[b]).wait()

        def compute_chunk(j, b):
            u_v, r_v = u_bufs[b], r_bufs[b]
            out0 = j * C

            def edge_body(i, c2):
                e4 = i * 4
                m = e4 % L
                contrib = jnp.zeros((L,), jnp.float32)
                for t in range(4):
                    e = e4 + t
                    p = [u_v[e, pl.ds(kk * L, L)] * r_v[e, pl.ds(kk * L, L)]
                         for kk in range(D // L)]
                    acc = ((p[0] + p[1]) + (p[2] + p[3])) + \
                          ((p[4] + p[5]) + (p[6] + p[7]))
                    for pm in perms:
                        acc = acc + acc.at[pm].get(mode="promise_in_bounds")
                    contrib = jnp.where(lane == m + t, acc, contrib)
                plsc.addupdate(o_v.at[pl.ds(out0 + e4 - m, L)], contrib)
                return c2

            lax.fori_loop(0, C // 4, edge_body, 0)

        for b in range(NBUF - 1):
            if b < n_chunks:
                start_gathers(b, b)

        def outer_body(g, c2):
            for b in range(NBUF):
                j = g * NBUF + b
                jn = j + NBUF - 1
                sb = b
                nb = (b + NBUF - 1) % NBUF

                @pl.when(jn < n_chunks)
                def _():
                    start_gathers(jn, nb)

                @pl.when(j < n_chunks)
                def _():
                    wait_gathers(j, sb)
                    compute_chunk(j, sb)

            return c2

        lax.fori_loop(0, n_outer, outer_body, 0)

        pltpu.sync_copy(o_v, out_hbm.at[pl.ds(base, per_w)])

    return k


def kernel(x_user, x_restaurant, edge_label_index):
    E = edge_label_index.shape[1]
    info = plsc.get_sparse_core_info()
    NC, NS = info.num_cores, info.num_subcores
    per_w = E // (NC * NS)
    n_chunks = per_w // C
    idx0 = edge_label_index[0].astype(jnp.int32)
    idx1 = edge_label_index[1].astype(jnp.int32)
    sc = _make_sc_kernel(E, NC, NS)
    return sc(x_user, x_restaurant, idx0, idx1)


# speedup vs baseline: 11.2484x; 1.1742x over previous
import functools

import jax
import jax.numpy as jnp
from jax import lax
from jax.experimental import pallas as pl
from jax.experimental.pallas import tpu as pltpu
from jax.experimental.pallas import tpu_sc as plsc

D = 128
L = 16
C = 80
NBUF = 4


def _make_sc_kernel(E, NC, NS):
    NW = NC * NS
    per_w = E // NW
    n_chunks = per_w // C
    n_outer = (n_chunks + NBUF - 1) // NBUF
    mesh = plsc.VectorSubcoreMesh(core_axis_name="c", subcore_axis_name="s")

    @functools.partial(
        pl.kernel,
        mesh=mesh,
        out_type=jax.ShapeDtypeStruct((E,), jnp.float32),
        scratch_types=(
            [pltpu.VMEM((per_w,), jnp.int32) for _ in range(2)]
            + [pltpu.VMEM((C, D), jnp.float32) for _ in range(2 * NBUF)]
            + [pltpu.VMEM((per_w,), jnp.float32)]
            + [pltpu.SemaphoreType.DMA for _ in range(NBUF)]
        ),
    )
    def k(xu_hbm, xr_hbm, i0_hbm, i1_hbm, out_hbm, *refs):
        i0_v, i1_v = refs[0], refs[1]
        u_bufs = refs[2:2 + NBUF]
        r_bufs = refs[2 + NBUF:2 + 2 * NBUF]
        o_v = refs[2 + 2 * NBUF]
        sems = refs[3 + 2 * NBUF:3 + 2 * NBUF + NBUF]

        wid = lax.axis_index("s") * NC + lax.axis_index("c")
        base = wid * per_w
        lane = lax.iota(jnp.int32, L)
        perms = [lane ^ (1 << t) for t in range(4)]

        pltpu.sync_copy(i0_hbm.at[pl.ds(base, per_w)], i0_v)
        pltpu.sync_copy(i1_hbm.at[pl.ds(base, per_w)], i1_v)

        def zero_body(g, c2):
            o_v[pl.ds(g * L, L)] = jnp.zeros((L,), jnp.float32)
            return c2

        lax.fori_loop(0, per_w // L, zero_body, 0)

        def start_gathers(j, b):
            pltpu.async_copy(xu_hbm.at[i0_v.at[pl.ds(j * C, C)]], u_bufs[b], sems[b])
            pltpu.async_copy(xr_hbm.at[i1_v.at[pl.ds(j * C, C)]], r_bufs[b], sems[b])

        def wait_gathers(j, b):
            pltpu.make_async_copy(xu_hbm.at[i0_v.at[pl.ds(j * C, C)]], u_bufs[b], sems[b]).wait()
            pltpu.make_async_copy(xr_hbm.at[i1_v.at[pl.ds(j * C, C)]], r_bufs[b], sems[b]).wait()

        def compute_chunk(j, b):
            u_v, r_v = u_bufs[b], r_bufs[b]
            out0 = j * C

            @plsc.parallel_loop(0, C // 4)
            def _(i):
                e4 = i * 4
                m = e4 % L
                contrib = jnp.zeros((L,), jnp.float32)
                for t in range(4):
                    e = e4 + t
                    p = [u_v[e, pl.ds(kk * L, L)] * r_v[e, pl.ds(kk * L, L)]
                         for kk in range(D // L)]
                    acc = ((p[0] + p[1]) + (p[2] + p[3])) + \
                          ((p[4] + p[5]) + (p[6] + p[7]))
                    for pm in perms:
                        acc = acc + acc.at[pm].get(mode="promise_in_bounds")
                    contrib = jnp.where(lane == m + t, acc, contrib)
                plsc.addupdate(o_v.at[pl.ds(out0 + e4 - m, L)], contrib)

        for b in range(NBUF - 1):
            if b < n_chunks:
                start_gathers(b, b)

        def outer_body(g, c2):
            for b in range(NBUF):
                j = g * NBUF + b
                jn = j + NBUF - 1
                sb = b
                nb = (b + NBUF - 1) % NBUF

                @pl.when(jn < n_chunks)
                def _():
                    start_gathers(jn, nb)

                @pl.when(j < n_chunks)
                def _():
                    wait_gathers(j, sb)
                    compute_chunk(j, sb)

            return c2

        lax.fori_loop(0, n_outer, outer_body, 0)

        pltpu.sync_copy(o_v, out_hbm.at[pl.ds(base, per_w)])

    return k


def kernel(x_user, x_restaurant, edge_label_index):
    E = edge_label_index.shape[1]
    info = plsc.get_sparse_core_info()
    NC, NS = info.num_cores, info.num_subcores
    per_w = E // (NC * NS)
    n_chunks = per_w // C
    idx0 = edge_label_index[0].astype(jnp.int32)
    idx1 = edge_label_index[1].astype(jnp.int32)
    sc = _make_sc_kernel(E, NC, NS)
    return sc(x_user, x_restaurant, idx0, idx1)
